# Initial kernel scaffold; baseline (speedup 1.0000x reference)
import jax
import jax.numpy as jnp
from jax.experimental import pallas as pl


def kernel(x, edge_index, W, b):
    raise NotImplementedError("write your pallas kernel here")


# speedup vs baseline: 3.1022x; 3.1022x over previous
import functools

import jax
import jax.numpy as jnp
from jax import lax
from jax.experimental import pallas as pl
from jax.experimental.pallas import tpu as pltpu
from jax.experimental.pallas import tpu_sc as plsc

NUM_CORES = 2
NUM_SUBCORES = 16
NUM_WORKERS = NUM_CORES * NUM_SUBCORES
LANES = 16
CHUNK = 80

_GATHER_DN = lax.GatherDimensionNumbers(
    offset_dims=(), collapsed_slice_dims=(0,), start_index_map=(0,))


def _vgather(v, idx):
    return lax.gather(v, idx[:, None], _GATHER_DN, slice_sizes=(1,),
                      mode=lax.GatherScatterMode.PROMISE_IN_BOUNDS)


def _sc_body(x_hbm, src_hbm, dst_hbm, wb_hbm, out_hbm,
             wb_v, src_idx, dst_idx, src_rows, dst_rows, out_buf,
             sem_a, sem_b):
    d = x_hbm.shape[1]
    n_dreg = d // LANES
    wid = lax.axis_index("s") * NUM_CORES + lax.axis_index("c")
    e_per_w = src_hbm.shape[0] // NUM_WORKERS
    base = wid * e_per_w
    n_chunks = e_per_w // CHUNK

    pltpu.sync_copy(wb_hbm, wb_v)
    b_vec = wb_v[pl.ds(d, LANES)]
    lane_iota = lax.iota(jnp.int32, LANES)

    def chunk_body(i, carry):
        cbase = base + i * CHUNK
        pltpu.sync_copy(src_hbm.at[pl.ds(cbase, CHUNK)], src_idx)
        pltpu.sync_copy(dst_hbm.at[pl.ds(cbase, CHUNK)], dst_idx)
        cp_s = pltpu.async_copy(x_hbm.at[src_idx], src_rows, sem_a)
        cp_d = pltpu.async_copy(x_hbm.at[dst_idx], dst_rows, sem_b)
        cp_s.wait()
        cp_d.wait()

        def group_body(g, c):
            ebase = g * LANES
            res = jnp.zeros((LANES,), jnp.float32)
            for j in range(LANES):
                e = ebase + j
                acc = (src_rows[e, pl.ds(0, LANES)]
                       * dst_rows[e, pl.ds(0, LANES)]) * wb_v[pl.ds(0, LANES)]
                for k in range(1, n_dreg):
                    acc = acc + (src_rows[e, pl.ds(k * LANES, LANES)]
                                 * dst_rows[e, pl.ds(k * LANES, LANES)]
                                 ) * wb_v[pl.ds(k * LANES, LANES)]
                for sh in (8, 4, 2, 1):
                    acc = acc + _vgather(acc, lane_iota ^ sh)
                res = jnp.where(lane_iota == j, acc, res)
            out_buf[pl.ds(g * LANES, LANES)] = res + b_vec
            return c

        lax.fori_loop(0, CHUNK // LANES, group_body, 0, unroll=False)
        pltpu.sync_copy(out_buf, out_hbm.at[pl.ds(cbase, CHUNK)])
        return carry

    lax.fori_loop(0, n_chunks, chunk_body, 0, unroll=False)


def _make_sc_call(n_edges, d):
    mesh = plsc.VectorSubcoreMesh(core_axis_name="c", subcore_axis_name="s")
    return pl.kernel(
        _sc_body,
        out_type=jax.ShapeDtypeStruct((n_edges,), jnp.float32),
        mesh=mesh,
        scratch_types=[
            pltpu.VMEM((d + LANES,), jnp.float32),
            pltpu.VMEM((CHUNK,), jnp.int32),
            pltpu.VMEM((CHUNK,), jnp.int32),
            pltpu.VMEM((CHUNK, d), jnp.float32),
            pltpu.VMEM((CHUNK, d), jnp.float32),
            pltpu.VMEM((CHUNK,), jnp.float32),
            pltpu.SemaphoreType.DMA,
            pltpu.SemaphoreType.DMA,
        ],
    )


def kernel(x, edge_index, W, b):
    n_edges = edge_index.shape[1]
    d = x.shape[1]
    src = edge_index[0].astype(jnp.int32)
    dst = edge_index[1].astype(jnp.int32)
    wb = jnp.concatenate(
        [W[0].astype(jnp.float32),
         jnp.broadcast_to(b.astype(jnp.float32), (LANES,))])
    out = _make_sc_call(n_edges, d)(x, src, dst, wb)
    return out.reshape(n_edges, 1)


# speedup vs baseline: 4.3290x; 1.3955x over previous
import functools

import jax
import jax.numpy as jnp
from jax import lax
from jax.experimental import pallas as pl
from jax.experimental.pallas import tpu as pltpu
from jax.experimental.pallas import tpu_sc as plsc

NUM_CORES = 2
NUM_SUBCORES = 16
NUM_WORKERS = NUM_CORES * NUM_SUBCORES
LANES = 16
CHUNK = 80

_GATHER_DN = lax.GatherDimensionNumbers(
    offset_dims=(), collapsed_slice_dims=(0,), start_index_map=(0,))


def _vgather(v, idx):
    return lax.gather(v, idx[:, None], _GATHER_DN, slice_sizes=(1,),
                      mode=lax.GatherScatterMode.PROMISE_IN_BOUNDS)


def _sc_body(x_hbm, src_hbm, dst_hbm, wb_hbm, out_hbm,
             wb_v, src_ix, dst_ix, out_all,
             srows0, drows0, srows1, drows1, sem0, sem1):
    d = x_hbm.shape[1]
    n_dreg = d // LANES
    wid = lax.axis_index("s") * NUM_CORES + lax.axis_index("c")
    e_per_w = src_hbm.shape[0] // NUM_WORKERS
    base = wid * e_per_w
    n_chunks = e_per_w // CHUNK

    pltpu.sync_copy(wb_hbm, wb_v)
    pltpu.sync_copy(src_hbm.at[pl.ds(base, e_per_w)], src_ix)
    pltpu.sync_copy(dst_hbm.at[pl.ds(base, e_per_w)], dst_ix)

    b_vec = wb_v[pl.ds(d, LANES)]
    w_regs = [wb_v[pl.ds(k * LANES, LANES)] for k in range(n_dreg)]
    lane_iota = lax.iota(jnp.int32, LANES)

    def issue(i, srows, drows, sem):
        sl = pl.ds(i * CHUNK, CHUNK)
        pltpu.async_copy(x_hbm.at[src_ix.at[sl]], srows, sem)
        pltpu.async_copy(x_hbm.at[dst_ix.at[sl]], drows, sem)

    def wait(i, srows, drows, sem):
        sl = pl.ds(i * CHUNK, CHUNK)
        pltpu.make_async_copy(x_hbm.at[src_ix.at[sl]], srows, sem).wait()
        pltpu.make_async_copy(x_hbm.at[dst_ix.at[sl]], drows, sem).wait()

    def compute(i, srows, drows):
        obase = i * CHUNK

        def group_body(g, c):
            ebase = g * LANES
            res = jnp.zeros((LANES,), jnp.float32)
            for j in range(LANES):
                e = ebase + j
                acc = (srows[e, pl.ds(0, LANES)]
                       * drows[e, pl.ds(0, LANES)]) * w_regs[0]
                for k in range(1, n_dreg):
                    acc = acc + (srows[e, pl.ds(k * LANES, LANES)]
                                 * drows[e, pl.ds(k * LANES, LANES)]
                                 ) * w_regs[k]
                for sh in (8, 4, 2, 1):
                    acc = acc + _vgather(acc, lane_iota ^ sh)
                res = jnp.where(lane_iota == j, acc, res)
            out_all[pl.ds(obase + ebase, LANES)] = res + b_vec
            return c

        lax.fori_loop(0, CHUNK // LANES, group_body, 0, unroll=False)

    issue(0, srows0, drows0, sem0)

    def pair_body(p, carry):
        c0 = 2 * p
        c1 = c0 + 1
        issue(c1, srows1, drows1, sem1)
        wait(c0, srows0, drows0, sem0)
        compute(c0, srows0, drows0)

        @pl.when(c1 + 1 < n_chunks)
        def _():
            issue(c1 + 1, srows0, drows0, sem0)

        wait(c1, srows1, drows1, sem1)
        compute(c1, srows1, drows1)
        return carry

    lax.fori_loop(0, n_chunks // 2, pair_body, 0, unroll=False)

    if n_chunks % 2 == 1:
        wait(n_chunks - 1, srows0, drows0, sem0)
        compute(n_chunks - 1, srows0, drows0)

    pltpu.sync_copy(out_all, out_hbm.at[pl.ds(base, e_per_w)])


def _make_sc_call(n_edges, d):
    mesh = plsc.VectorSubcoreMesh(core_axis_name="c", subcore_axis_name="s")
    e_per_w = n_edges // NUM_WORKERS
    return pl.kernel(
        _sc_body,
        out_type=jax.ShapeDtypeStruct((n_edges,), jnp.float32),
        mesh=mesh,
        scratch_types=[
            pltpu.VMEM((d + LANES,), jnp.float32),
            pltpu.VMEM((e_per_w,), jnp.int32),
            pltpu.VMEM((e_per_w,), jnp.int32),
            pltpu.VMEM((e_per_w,), jnp.float32),
            pltpu.VMEM((CHUNK, d), jnp.float32),
            pltpu.VMEM((CHUNK, d), jnp.float32),
            pltpu.VMEM((CHUNK, d), jnp.float32),
            pltpu.VMEM((CHUNK, d), jnp.float32),
            pltpu.SemaphoreType.DMA,
            pltpu.SemaphoreType.DMA,
        ],
    )


def kernel(x, edge_index, W, b):
    n_edges = edge_index.shape[1]
    d = x.shape[1]
    src = edge_index[0].astype(jnp.int32)
    dst = edge_index[1].astype(jnp.int32)
    wb = jnp.concatenate(
        [W[0].astype(jnp.float32),
         jnp.broadcast_to(b.astype(jnp.float32), (LANES,))])
    out = _make_sc_call(n_edges, d)(x, src, dst, wb)
    return out.reshape(n_edges, 1)


# speedup vs baseline: 5.3641x; 1.2391x over previous
import functools

import jax
import jax.numpy as jnp
from jax import lax
from jax.experimental import pallas as pl
from jax.experimental.pallas import tpu as pltpu
from jax.experimental.pallas import tpu_sc as plsc

NUM_CORES = 2
NUM_SUBCORES = 16
NUM_WORKERS = NUM_CORES * NUM_SUBCORES
LANES = 16
CHUNK = 80

_GATHER_DN = lax.GatherDimensionNumbers(
    offset_dims=(), collapsed_slice_dims=(0,), start_index_map=(0,))


def _vgather(v, idx):
    return lax.gather(v, idx[:, None], _GATHER_DN, slice_sizes=(1,),
                      mode=lax.GatherScatterMode.PROMISE_IN_BOUNDS)


def _scale_body(x_ref, w_ref, y_ref):
    y_ref[...] = x_ref[...] * w_ref[...]


def _scale_rows(x, w_row):
    n, d = x.shape
    blk = 1000
    return pl.pallas_call(
        _scale_body,
        out_shape=jax.ShapeDtypeStruct((n, d), jnp.float32),
        grid=(n // blk,),
        in_specs=[pl.BlockSpec((blk, d), lambda i: (i, 0)),
                  pl.BlockSpec((1, d), lambda i: (0, 0))],
        out_specs=pl.BlockSpec((blk, d), lambda i: (i, 0)),
    )(x, w_row)


def _sc_body(x_hbm, y_hbm, src_hbm, dst_hbm, b_hbm, out_hbm,
             b_v, src_ix, dst_ix, out_all,
             srows0, drows0, srows1, drows1, sem0, sem1):
    d = x_hbm.shape[1]
    n_dreg = d // LANES
    wid = lax.axis_index("s") * NUM_CORES + lax.axis_index("c")
    e_per_w = src_hbm.shape[0] // NUM_WORKERS
    base = wid * e_per_w
    n_chunks = e_per_w // CHUNK

    pltpu.sync_copy(b_hbm, b_v)
    pltpu.sync_copy(src_hbm.at[pl.ds(base, e_per_w)], src_ix)
    pltpu.sync_copy(dst_hbm.at[pl.ds(base, e_per_w)], dst_ix)

    b_vec = b_v[...]
    lane_iota = lax.iota(jnp.int32, LANES)

    def issue(i, srows, drows, sem):
        sl = pl.ds(i * CHUNK, CHUNK)
        pltpu.async_copy(x_hbm.at[src_ix.at[sl]], srows, sem)
        pltpu.async_copy(y_hbm.at[dst_ix.at[sl]], drows, sem)

    def wait(i, srows, drows, sem):
        sl = pl.ds(i * CHUNK, CHUNK)
        pltpu.make_async_copy(x_hbm.at[src_ix.at[sl]], srows, sem).wait()
        pltpu.make_async_copy(y_hbm.at[dst_ix.at[sl]], drows, sem).wait()

    def compute(i, srows, drows):
        obase = i * CHUNK

        def group_body(g, c):
            ebase = g * LANES
            accs = []
            for j in range(LANES):
                e = ebase + j
                ps = [srows[e, pl.ds(k * LANES, LANES)]
                      * drows[e, pl.ds(k * LANES, LANES)]
                      for k in range(n_dreg)]
                while len(ps) > 1:
                    ps = [ps[t] + ps[t + 1] for t in range(0, len(ps), 2)]
                accs.append(ps[0])
            for dd in (1, 2, 4, 8):
                m = (lane_iota & dd) != 0
                nxt = []
                for t in range(0, len(accs), 2):
                    a, bb = accs[t], accs[t + 1]
                    sel = jnp.where(m, bb, a)
                    rot = jnp.where(m, _vgather(bb, lane_iota ^ dd),
                                    _vgather(a, lane_iota ^ dd))
                    nxt.append(sel + rot)
                accs = nxt
            out_all[pl.ds(obase + ebase, LANES)] = accs[0] + b_vec
            return c

        lax.fori_loop(0, CHUNK // LANES, group_body, 0, unroll=False)

    issue(0, srows0, drows0, sem0)

    def pair_body(p, carry):
        c0 = 2 * p
        c1 = c0 + 1
        issue(c1, srows1, drows1, sem1)
        wait(c0, srows0, drows0, sem0)
        compute(c0, srows0, drows0)

        @pl.when(c1 + 1 < n_chunks)
        def _():
            issue(c1 + 1, srows0, drows0, sem0)

        wait(c1, srows1, drows1, sem1)
        compute(c1, srows1, drows1)
        return carry

    lax.fori_loop(0, n_chunks // 2, pair_body, 0, unroll=False)

    if n_chunks % 2 == 1:
        wait(n_chunks - 1, srows0, drows0, sem0)
        compute(n_chunks - 1, srows0, drows0)

    pltpu.sync_copy(out_all, out_hbm.at[pl.ds(base, e_per_w)])


def _make_sc_call(n_edges, d):
    mesh = plsc.VectorSubcoreMesh(core_axis_name="c", subcore_axis_name="s")
    e_per_w = n_edges // NUM_WORKERS
    return pl.kernel(
        _sc_body,
        out_type=jax.ShapeDtypeStruct((n_edges,), jnp.float32),
        mesh=mesh,
        scratch_types=[
            pltpu.VMEM((LANES,), jnp.float32),
            pltpu.VMEM((e_per_w,), jnp.int32),
            pltpu.VMEM((e_per_w,), jnp.int32),
            pltpu.VMEM((e_per_w,), jnp.float32),
            pltpu.VMEM((CHUNK, d), jnp.float32),
            pltpu.VMEM((CHUNK, d), jnp.float32),
            pltpu.VMEM((CHUNK, d), jnp.float32),
            pltpu.VMEM((CHUNK, d), jnp.float32),
            pltpu.SemaphoreType.DMA,
            pltpu.SemaphoreType.DMA,
        ],
    )


def kernel(x, edge_index, W, b):
    n_edges = edge_index.shape[1]
    d = x.shape[1]
    src = edge_index[0].astype(jnp.int32)
    dst = edge_index[1].astype(jnp.int32)
    y = _scale_rows(x, W.astype(jnp.float32))
    b16 = jnp.broadcast_to(b.astype(jnp.float32), (LANES,))
    out = _make_sc_call(n_edges, d)(x, y, src, dst, b16)
    return out.reshape(n_edges, 1)


# speedup vs baseline: 8.1984x; 1.5284x over previous
import functools

import jax
import jax.numpy as jnp
from jax import lax
from jax.experimental import pallas as pl
from jax.experimental.pallas import tpu as pltpu
from jax.experimental.pallas import tpu_sc as plsc

NUM_CORES = 2
NUM_SUBCORES = 16
NUM_WORKERS = NUM_CORES * NUM_SUBCORES
LANES = 16
CHUNK = 80

_GATHER_DN = lax.GatherDimensionNumbers(
    offset_dims=(), collapsed_slice_dims=(0,), start_index_map=(0,))


def _vgather(v, idx):
    return lax.gather(v, idx[:, None], _GATHER_DN, slice_sizes=(1,),
                      mode=lax.GatherScatterMode.PROMISE_IN_BOUNDS)


def _scale_body(x_ref, w_ref, y_ref):
    y_ref[...] = x_ref[...] * w_ref[...]


def _scale_rows(x, w_row):
    n, d = x.shape
    blk = 1000
    return pl.pallas_call(
        _scale_body,
        out_shape=jax.ShapeDtypeStruct((n, d), jnp.float32),
        grid=(n // blk,),
        in_specs=[pl.BlockSpec((blk, d), lambda i: (i, 0)),
                  pl.BlockSpec((1, d), lambda i: (0, 0))],
        out_specs=pl.BlockSpec((blk, d), lambda i: (i, 0)),
    )(x, w_row)


def _sc_body(x_hbm, y_hbm, src_hbm, dst_hbm, b_hbm, out_hbm,
             b_v, src_ix, dst_ix, out_all,
             srows0, drows0, srows1, drows1, sem0, sem1):
    n_sub = x_hbm.shape[1] // LANES
    wid = lax.axis_index("s") * NUM_CORES + lax.axis_index("c")
    e_per_w = src_hbm.shape[0] // NUM_WORKERS
    base = wid * e_per_w
    n_chunks = e_per_w // CHUNK

    pltpu.sync_copy(b_hbm, b_v)
    pltpu.sync_copy(src_hbm.at[pl.ds(base, e_per_w)], src_ix)
    pltpu.sync_copy(dst_hbm.at[pl.ds(base, e_per_w)], dst_ix)

    b_vec = b_v[...]
    lane_iota = lax.iota(jnp.int32, LANES)

    def issue(i, srows, drows, sem):
        sl = pl.ds(i * CHUNK, CHUNK)
        pltpu.async_copy(x_hbm.at[src_ix.at[sl]], srows, sem)
        pltpu.async_copy(y_hbm.at[dst_ix.at[sl]], drows, sem)

    def wait(i, srows, drows, sem):
        sl = pl.ds(i * CHUNK, CHUNK)
        pltpu.make_async_copy(x_hbm.at[src_ix.at[sl]], srows, sem).wait()
        pltpu.make_async_copy(y_hbm.at[dst_ix.at[sl]], drows, sem).wait()

    def compute(i, srows, drows):
        obase = i * CHUNK

        def group_body(g, c):
            ebase = g * LANES
            init = tuple(srows[ebase + j, pl.ds(0, LANES)]
                         * drows[ebase + j, pl.ds(0, LANES)]
                         for j in range(LANES))

            def kbody(k, accs):
                off = k * LANES
                return tuple(
                    accs[j] + (srows[ebase + j, pl.ds(off, LANES)]
                               * drows[ebase + j, pl.ds(off, LANES)])
                    for j in range(LANES))

            accs = list(lax.fori_loop(1, n_sub, kbody, init, unroll=False))
            for dd in (1, 2, 4, 8):
                m = (lane_iota & dd) != 0
                nxt = []
                for t in range(0, len(accs), 2):
                    a, bb = accs[t], accs[t + 1]
                    sel = jnp.where(m, bb, a)
                    rot = jnp.where(m, _vgather(bb, lane_iota ^ dd),
                                    _vgather(a, lane_iota ^ dd))
                    nxt.append(sel + rot)
                accs = nxt
            out_all[pl.ds(obase + ebase, LANES)] = accs[0] + b_vec
            return c

        lax.fori_loop(0, CHUNK // LANES, group_body, 0, unroll=False)

    issue(0, srows0, drows0, sem0)

    def pair_body(p, carry):
        c0 = 2 * p
        c1 = c0 + 1
        issue(c1, srows1, drows1, sem1)
        wait(c0, srows0, drows0, sem0)
        compute(c0, srows0, drows0)

        @pl.when(c1 + 1 < n_chunks)
        def _():
            issue(c1 + 1, srows0, drows0, sem0)

        wait(c1, srows1, drows1, sem1)
        compute(c1, srows1, drows1)
        return carry

    lax.fori_loop(0, n_chunks // 2, pair_body, 0, unroll=False)

    if n_chunks % 2 == 1:
        wait(n_chunks - 1, srows0, drows0, sem0)
        compute(n_chunks - 1, srows0, drows0)

    pltpu.sync_copy(out_all, out_hbm.at[pl.ds(base, e_per_w)])


def _make_sc_call(n_edges, d):
    mesh = plsc.VectorSubcoreMesh(core_axis_name="c", subcore_axis_name="s")
    e_per_w = n_edges // NUM_WORKERS
    return pl.kernel(
        _sc_body,
        out_type=jax.ShapeDtypeStruct((n_edges,), jnp.float32),
        mesh=mesh,
        scratch_types=[
            pltpu.VMEM((LANES,), jnp.float32),
            pltpu.VMEM((e_per_w,), jnp.int32),
            pltpu.VMEM((e_per_w,), jnp.int32),
            pltpu.VMEM((e_per_w,), jnp.float32),
            pltpu.VMEM((CHUNK, d), jnp.float32),
            pltpu.VMEM((CHUNK, d), jnp.float32),
            pltpu.VMEM((CHUNK, d), jnp.float32),
            pltpu.VMEM((CHUNK, d), jnp.float32),
            pltpu.SemaphoreType.DMA,
            pltpu.SemaphoreType.DMA,
        ],
    )


def kernel(x, edge_index, W, b):
    n_edges = edge_index.shape[1]
    d = x.shape[1]
    src = edge_index[0].astype(jnp.int32)
    dst = edge_index[1].astype(jnp.int32)
    y = _scale_rows(x, W.astype(jnp.float32))
    b16 = jnp.broadcast_to(b.astype(jnp.float32), (LANES,))
    out = _make_sc_call(n_edges, d)(x, y, src, dst, b16)
    return out.reshape(n_edges, 1)


# speedup vs baseline: 8.4500x; 1.0307x over previous
import functools

import jax
import jax.numpy as jnp
from jax import lax
from jax.experimental import pallas as pl
from jax.experimental.pallas import tpu as pltpu
from jax.experimental.pallas import tpu_sc as plsc

NUM_CORES = 2
NUM_SUBCORES = 16
NUM_WORKERS = NUM_CORES * NUM_SUBCORES
LANES = 16
CHUNK = 80

_GATHER_DN = lax.GatherDimensionNumbers(
    offset_dims=(), collapsed_slice_dims=(0,), start_index_map=(0,))


def _vgather(v, idx):
    return lax.gather(v, idx[:, None], _GATHER_DN, slice_sizes=(1,),
                      mode=lax.GatherScatterMode.PROMISE_IN_BOUNDS)


def _sc_body(x_hbm, src_hbm, dst_hbm, wb_hbm, out_hbm,
             wb_v, src_ix, dst_ix, out_all,
             srows0, drows0, srows1, drows1, sem0, sem1):
    d = x_hbm.shape[1]
    n_sub = d // LANES
    wid = lax.axis_index("s") * NUM_CORES + lax.axis_index("c")
    e_per_w = src_hbm.shape[0] // NUM_WORKERS
    base = wid * e_per_w
    n_chunks = e_per_w // CHUNK

    pltpu.sync_copy(wb_hbm, wb_v)
    pltpu.sync_copy(src_hbm.at[pl.ds(base, e_per_w)], src_ix)
    pltpu.sync_copy(dst_hbm.at[pl.ds(base, e_per_w)], dst_ix)

    b_vec = wb_v[pl.ds(d, LANES)]
    lane_iota = lax.iota(jnp.int32, LANES)

    def issue(i, srows, drows, sem):
        sl = pl.ds(i * CHUNK, CHUNK)
        pltpu.async_copy(x_hbm.at[src_ix.at[sl]], srows, sem)
        pltpu.async_copy(x_hbm.at[dst_ix.at[sl]], drows, sem)

    def wait(i, srows, drows, sem):
        sl = pl.ds(i * CHUNK, CHUNK)
        pltpu.make_async_copy(x_hbm.at[src_ix.at[sl]], srows, sem).wait()
        pltpu.make_async_copy(x_hbm.at[dst_ix.at[sl]], drows, sem).wait()

    def compute(i, srows, drows):
        obase = i * CHUNK

        def group_body(g, c):
            ebase = g * LANES
            w0 = wb_v[pl.ds(0, LANES)]
            init = tuple(srows[ebase + j, pl.ds(0, LANES)]
                         * drows[ebase + j, pl.ds(0, LANES)] * w0
                         for j in range(LANES))

            def kbody(k, accs):
                off = k * LANES
                wk = wb_v[pl.ds(off, LANES)]
                return tuple(
                    accs[j] + (srows[ebase + j, pl.ds(off, LANES)]
                               * drows[ebase + j, pl.ds(off, LANES)]) * wk
                    for j in range(LANES))

            accs = list(lax.fori_loop(1, n_sub, kbody, init, unroll=False))
            for dd in (1, 2, 4, 8):
                m = (lane_iota & dd) != 0
                rot_idx = lane_iota ^ dd
                nxt = []
                for t in range(0, len(accs), 2):
                    a, bb = accs[t], accs[t + 1]
                    sel = jnp.where(m, bb, a)
                    rot = _vgather(jnp.where(m, a, bb), rot_idx)
                    nxt.append(sel + rot)
                accs = nxt
            out_all[pl.ds(obase + ebase, LANES)] = accs[0] + b_vec
            return c

        lax.fori_loop(0, CHUNK // LANES, group_body, 0, unroll=False)

    issue(0, srows0, drows0, sem0)

    def pair_body(p, carry):
        c0 = 2 * p
        c1 = c0 + 1
        issue(c1, srows1, drows1, sem1)
        wait(c0, srows0, drows0, sem0)
        compute(c0, srows0, drows0)

        @pl.when(c1 + 1 < n_chunks)
        def _():
            issue(c1 + 1, srows0, drows0, sem0)

        wait(c1, srows1, drows1, sem1)
        compute(c1, srows1, drows1)
        return carry

    lax.fori_loop(0, n_chunks // 2, pair_body, 0, unroll=False)

    if n_chunks % 2 == 1:
        wait(n_chunks - 1, srows0, drows0, sem0)
        compute(n_chunks - 1, srows0, drows0)

    pltpu.sync_copy(out_all, out_hbm.at[pl.ds(base, e_per_w)])


def _make_sc_call(n_edges, d):
    mesh = plsc.VectorSubcoreMesh(core_axis_name="c", subcore_axis_name="s")
    e_per_w = n_edges // NUM_WORKERS
    return pl.kernel(
        _sc_body,
        out_type=jax.ShapeDtypeStruct((n_edges,), jnp.float32),
        mesh=mesh,
        scratch_types=[
            pltpu.VMEM((d + LANES,), jnp.float32),
            pltpu.VMEM((e_per_w,), jnp.int32),
            pltpu.VMEM((e_per_w,), jnp.int32),
            pltpu.VMEM((e_per_w,), jnp.float32),
            pltpu.VMEM((CHUNK, d), jnp.float32),
            pltpu.VMEM((CHUNK, d), jnp.float32),
            pltpu.VMEM((CHUNK, d), jnp.float32),
            pltpu.VMEM((CHUNK, d), jnp.float32),
            pltpu.SemaphoreType.DMA,
            pltpu.SemaphoreType.DMA,
        ],
    )


def kernel(x, edge_index, W, b):
    n_edges = edge_index.shape[1]
    d = x.shape[1]
    src = edge_index[0].astype(jnp.int32)
    dst = edge_index[1].astype(jnp.int32)
    wb = jnp.concatenate(
        [W[0].astype(jnp.float32),
         jnp.broadcast_to(b.astype(jnp.float32), (LANES,))])
    out = _make_sc_call(n_edges, d)(x, src, dst, wb)
    return out.reshape(n_edges, 1)
